# baseline (speedup 1.0000x reference)
import jax
import jax.numpy as jnp
from jax.experimental import pallas as pl
from jax.experimental.pallas import tpu as pltpu

_TEMPERATURE = 45.254834
_LOG2E_OVER_T = 1.4426950408889634 / _TEMPERATURE


def _attn_kernel(q_ref, k_ref, o_ref, s_ref):
    i = pl.program_id(0)
    p = jax.lax.rem(i, 2)
    s_ref[p] = jax.lax.dot_general(
        q_ref[...],
        k_ref[...],
        (((1,), (1,)), ((), ())),
        preferred_element_type=jnp.float32,
    )
    x = s_ref[1 - p]
    m = jnp.max(x, axis=-1, keepdims=True)
    e = jnp.exp2(x - m)
    r = 1.0 / jnp.sum(e, axis=-1, keepdims=True)
    o_ref[...] = e * r


def kernel(q, k):
    n, d = q.shape
    nk = k.shape[0]
    br = 256
    ni = n // br
    qb = (q * _LOG2E_OVER_T).astype(jnp.bfloat16)
    kb = k.astype(jnp.bfloat16)
    return pl.pallas_call(
        _attn_kernel,
        grid=(ni + 1,),
        in_specs=[
            pl.BlockSpec((br, d), lambda i: (jnp.minimum(i, ni - 1), 0)),
            pl.BlockSpec((nk, d), lambda i: (0, 0)),
        ],
        out_specs=pl.BlockSpec((br, nk), lambda i: (jnp.maximum(i - 1, 0), 0)),
        out_shape=jax.ShapeDtypeStruct((n, nk), jnp.float32),
        scratch_shapes=[pltpu.VMEM((2, br, nk), jnp.float32)],
        compiler_params=pltpu.CompilerParams(
            dimension_semantics=("arbitrary",)
        ),
    )(qb, kb)


# speedup vs baseline: 1.0128x; 1.0128x over previous
import jax
import jax.numpy as jnp
from jax.experimental import pallas as pl
from jax.experimental.pallas import tpu as pltpu

_TEMPERATURE = 45.254834
_LOG2E_OVER_T = 1.4426950408889634 / _TEMPERATURE


def _dot(q_ref, k_ref):
    return jax.lax.dot_general(
        q_ref[...],
        k_ref[...],
        (((1,), (1,)), ((), ())),
        preferred_element_type=jnp.float32,
    )


def _softmax(x, o_ref):
    m = jnp.max(x, axis=-1, keepdims=True)
    e = jnp.exp2(x - m)
    r = 1.0 / jnp.sum(e, axis=-1, keepdims=True)
    o_ref[...] = e * r


def _attn_kernel(q_ref, k_ref, o_ref, a_ref, b_ref):
    i = pl.program_id(0)
    p = jax.lax.rem(i, 2)

    @pl.when(p == 0)
    def _even():
        a_ref[...] = _dot(q_ref, k_ref)
        _softmax(b_ref[...], o_ref)

    @pl.when(p == 1)
    def _odd():
        b_ref[...] = _dot(q_ref, k_ref)
        _softmax(a_ref[...], o_ref)


def kernel(q, k):
    n, d = q.shape
    nk = k.shape[0]
    br = 256
    ni = n // br
    qb = (q * _LOG2E_OVER_T).astype(jnp.bfloat16)
    kb = k.astype(jnp.bfloat16)
    return pl.pallas_call(
        _attn_kernel,
        grid=(ni + 1,),
        in_specs=[
            pl.BlockSpec((br, d), lambda i: (jnp.minimum(i, ni - 1), 0)),
            pl.BlockSpec((nk, d), lambda i: (0, 0)),
        ],
        out_specs=pl.BlockSpec((br, nk), lambda i: (jnp.maximum(i - 1, 0), 0)),
        out_shape=jax.ShapeDtypeStruct((n, nk), jnp.float32),
        scratch_shapes=[
            pltpu.VMEM((br, nk), jnp.float32),
            pltpu.VMEM((br, nk), jnp.float32),
        ],
        compiler_params=pltpu.CompilerParams(
            dimension_semantics=("arbitrary",)
        ),
    )(qb, kb)


# speedup vs baseline: 1.0596x; 1.0463x over previous
import jax
import jax.numpy as jnp
from jax.experimental import pallas as pl
from jax.experimental.pallas import tpu as pltpu

_TEMPERATURE = 45.254834
_LOG2E_OVER_T = 1.4426950408889634 / _TEMPERATURE


def _attn_kernel(q_ref, k_ref, o_ref):
    x = jax.lax.dot_general(
        q_ref[...],
        k_ref[...],
        (((1,), (1,)), ((), ())),
        preferred_element_type=jnp.float32,
    )
    m = jnp.max(x, axis=-1, keepdims=True)
    e = jnp.exp2(x - m)
    r = 1.0 / jnp.sum(e, axis=-1, keepdims=True)
    o_ref[...] = e * r


def kernel(q, k):
    n, d = q.shape
    nk = k.shape[0]
    br = 256
    qb = (q * _LOG2E_OVER_T).astype(jnp.bfloat16)
    kb = k.astype(jnp.bfloat16)
    return pl.pallas_call(
        _attn_kernel,
        grid=(n // br,),
        in_specs=[
            pl.BlockSpec((br, d), lambda i: (i, 0)),
            pl.BlockSpec((nk, d), lambda i: (0, 0)),
        ],
        out_specs=pl.BlockSpec((br, nk), lambda i: (i, 0)),
        out_shape=jax.ShapeDtypeStruct((n, nk), jnp.float32),
        compiler_params=pltpu.CompilerParams(
            dimension_semantics=("parallel",)
        ),
    )(qb, kb)


# speedup vs baseline: 1.2053x; 1.1374x over previous
import jax
import jax.numpy as jnp
from jax.experimental import pallas as pl
from jax.experimental.pallas import tpu as pltpu

_TEMPERATURE = 45.254834
_LOG2E_OVER_T = 1.4426950408889634 / _TEMPERATURE


def _attn_kernel(q_ref, k_ref, o_ref):
    qs = (q_ref[...] * _LOG2E_OVER_T).astype(jnp.bfloat16)
    x = jax.lax.dot_general(
        qs,
        k_ref[...],
        (((1,), (1,)), ((), ())),
        preferred_element_type=jnp.float32,
    )
    m = jnp.max(x, axis=-1, keepdims=True)
    e = jnp.exp2(x - m)
    r = 1.0 / jnp.sum(e, axis=-1, keepdims=True)
    o_ref[...] = e * r


def kernel(q, k):
    n, d = q.shape
    nk = k.shape[0]
    br = 256
    kb = k.astype(jnp.bfloat16)
    return pl.pallas_call(
        _attn_kernel,
        grid=(n // br,),
        in_specs=[
            pl.BlockSpec((br, d), lambda i: (i, 0)),
            pl.BlockSpec((nk, d), lambda i: (0, 0)),
        ],
        out_specs=pl.BlockSpec((br, nk), lambda i: (i, 0)),
        out_shape=jax.ShapeDtypeStruct((n, nk), jnp.float32),
        compiler_params=pltpu.CompilerParams(
            dimension_semantics=("parallel",)
        ),
    )(q, kb)


# speedup vs baseline: 1.3359x; 1.1084x over previous
import jax
import jax.numpy as jnp
from jax.experimental import pallas as pl
from jax.experimental.pallas import tpu as pltpu

_TEMPERATURE = 45.254834
_LOG2E_OVER_T = 1.4426950408889634 / _TEMPERATURE

_KCHUNK = 512


def _attn_kernel(q_ref, k_ref, o_ref, kb_ref, kf_ref, sem):
    i = pl.program_id(0)
    nk = kb_ref.shape[0]
    nchunks = nk // _KCHUNK

    @pl.when(i == 0)
    def _load_k():
        def copy(c, buf):
            return pltpu.make_async_copy(
                k_ref.at[pl.ds(c * _KCHUNK, _KCHUNK), :],
                kf_ref.at[buf],
                sem.at[buf],
            )

        copy(0, 0).start()
        for c in range(nchunks):
            if c + 1 < nchunks:
                copy(c + 1, (c + 1) % 2).start()
            copy(c, c % 2).wait()
            kb_ref[pl.ds(c * _KCHUNK, _KCHUNK), :] = (
                kf_ref[c % 2].astype(jnp.bfloat16)
            )

    qs = (q_ref[...] * _LOG2E_OVER_T).astype(jnp.bfloat16)
    x = jax.lax.dot_general(
        qs,
        kb_ref[...],
        (((1,), (1,)), ((), ())),
        preferred_element_type=jnp.float32,
    )
    m = jnp.max(x, axis=-1, keepdims=True)
    e = jnp.exp2(x - m)
    r = 1.0 / jnp.sum(e, axis=-1, keepdims=True)
    o_ref[...] = e * r


def kernel(q, k):
    n, d = q.shape
    nk = k.shape[0]
    br = 256
    return pl.pallas_call(
        _attn_kernel,
        grid=(n // br,),
        in_specs=[
            pl.BlockSpec((br, d), lambda i: (i, 0)),
            pl.BlockSpec(memory_space=pltpu.MemorySpace.HBM),
        ],
        out_specs=pl.BlockSpec((br, nk), lambda i: (i, 0)),
        out_shape=jax.ShapeDtypeStruct((n, nk), jnp.float32),
        scratch_shapes=[
            pltpu.VMEM((nk, d), jnp.bfloat16),
            pltpu.VMEM((2, _KCHUNK, d), jnp.float32),
            pltpu.SemaphoreType.DMA((2,)),
        ],
        compiler_params=pltpu.CompilerParams(
            dimension_semantics=("arbitrary",)
        ),
    )(q, k)
